# baseline (speedup 1.0000x reference)
import functools

import jax
import jax.numpy as jnp
from jax import lax
from jax.experimental import pallas as pl
from jax.experimental.pallas import tpu as pltpu
from jax.experimental.pallas import tpu_sc as plsc

V = 8192
N = 32

_NC = 2
_NS = 16


def _gather_body(table_hbm, x_hbm, out_hbm, xv, idxbuf, row, sem):
  c = lax.axis_index("c")
  s = lax.axis_index("s")
  w = c * _NS + s
  pltpu.sync_copy(x_hbm, xv)
  lanes = lax.iota(jnp.int32, 16)
  half = jnp.where(jnp.full((16,), c, jnp.int32) == 0,
                   xv[pl.ds(0, 16)], xv[pl.ds(16, 16)])
  plsc.store_compressed(idxbuf.at[pl.ds(0, 16)], half,
                        mask=lanes == jnp.full((16,), s, jnp.int32))
  pltpu.async_copy(table_hbm.at[idxbuf.at[pl.ds(0, 1)]], row, sem).wait()
  pltpu.sync_copy(row, out_hbm.at[pl.ds(w, 1)])


@functools.lru_cache(maxsize=1)
def _make_gather():
  return pl.kernel(
      _gather_body,
      mesh=plsc.VectorSubcoreMesh(
          core_axis_name="c", subcore_axis_name="s",
          num_cores=_NC, num_subcores=_NS),
      out_type=jax.ShapeDtypeStruct((N, V), jnp.float32),
      compiler_params=pltpu.CompilerParams(needs_layout_passes=False),
      scratch_types=[
          pltpu.VMEM((N,), jnp.int32),
          pltpu.VMEM((16,), jnp.int32),
          pltpu.VMEM((1, V), jnp.float32),
          pltpu.SemaphoreType.DMA,
      ],
  )


def _loss_body(x_sref, y_sref, row_ref, out_ref, acc_ref):
  i = pl.program_id(0)
  xi = x_sref[i]
  row = row_ref[pl.ds(jnp.bitwise_and(xi, 7), 1), :]
  m = jnp.max(row)
  lse = m + jnp.log(jnp.sum(jnp.exp(row - m)))
  ids = lax.broadcasted_iota(jnp.int32, (1, V), 1)
  yi = y_sref[jnp.right_shift(i, 3), jnp.bitwise_and(i, 7)]
  tgt = jnp.sum(jnp.where(ids == yi, row, 0.0))

  @pl.when(i == 0)
  def _():
    acc_ref[0] = 0.0

  acc_ref[0] += lse - tgt

  @pl.when(i == N - 1)
  def _():
    out_ref[0, 0] = acc_ref[0] * (1.0 / N)


@functools.lru_cache(maxsize=1)
def _make_loss():
  grid_spec = pltpu.PrefetchScalarGridSpec(
      num_scalar_prefetch=2,
      grid=(N,),
      in_specs=[
          pl.BlockSpec((8, V), lambda i, xs, ys: (xs[i] // 8, 0)),
      ],
      out_specs=pl.BlockSpec(memory_space=pltpu.SMEM),
      scratch_shapes=[pltpu.SMEM((1,), jnp.float32)],
  )
  return pl.pallas_call(
      _loss_body,
      grid_spec=grid_spec,
      out_shape=jax.ShapeDtypeStruct((1, 1), jnp.float32),
  )


def kernel(x, y, table):
  xf = x.reshape(N).astype(jnp.int32)
  logits = _make_gather()(table, xf)
  loss = _make_loss()(xf, y.astype(jnp.int32), table)[0, 0]
  return logits, loss
